# baseline (speedup 1.0000x reference)
import jax
import jax.numpy as jnp
from jax.experimental import pallas as pl
from jax.experimental.pallas import tpu as pltpu


def _body(B, T, mask_any, hidden_ref, out_ref, mask_v, rows_v, m_sem, g_sem, o_sem):
    m_copy = pltpu.make_async_copy(mask_any, mask_v, m_sem)
    m_copy.start()
    spec = []
    for b in range(B):
        c = pltpu.make_async_copy(
            hidden_ref.at[b, pl.ds(T - 1, 1), :],
            rows_v.at[pl.ds(b, 1), :],
            g_sem,
        )
        c.start()
        spec.append(c)
    m_copy.wait()
    lasts = [jnp.maximum(jnp.sum(mask_v[b, :]) - 1, 0) for b in range(B)]
    outs = []
    for b in range(B):
        spec[b].wait()
        o = pltpu.make_async_copy(
            rows_v.at[pl.ds(b, 1), :],
            out_ref.at[pl.ds(b, 1), :],
            o_sem,
        )
        o.start()
        outs.append(o)
    mispredicted = lasts[0] != T - 1
    for b in range(1, B):
        mispredicted = mispredicted | (lasts[b] != T - 1)

    for o in outs:
        o.wait()

    @pl.when(mispredicted)
    def _():
        for b in range(B):
            @pl.when(lasts[b] != T - 1)
            def _(b=b):
                fix = pltpu.make_async_copy(
                    hidden_ref.at[b, pl.ds(lasts[b], 1), :],
                    out_ref.at[pl.ds(b, 1), :],
                    g_sem,
                )
                fix.start()
                fix.wait()


def kernel(last_hidden_state, attention_mask):
    B, T, H = last_hidden_state.shape
    mask = attention_mask.astype(jnp.int32)
    return pl.pallas_call(
        lambda *refs: _body(B, T, *refs),
        out_shape=jax.ShapeDtypeStruct((B, H), jnp.float32),
        in_specs=[
            pl.BlockSpec(memory_space=pl.ANY),
            pl.BlockSpec(memory_space=pl.ANY),
        ],
        out_specs=pl.BlockSpec(memory_space=pl.ANY),
        scratch_shapes=[
            pltpu.VMEM((B, T), jnp.int32),
            pltpu.VMEM((B, H), jnp.float32),
            pltpu.SemaphoreType.DMA,
            pltpu.SemaphoreType.DMA,
            pltpu.SemaphoreType.DMA,
        ],
    )(mask, last_hidden_state)
